# baseline (speedup 1.0000x reference)
import functools

import jax
import jax.numpy as jnp
from jax import lax
from jax.experimental import pallas as pl
from jax.experimental.pallas import tpu as pltpu
from jax.experimental.pallas import tpu_sc as plsc

_VOCAB = 100000
_D = 64
_MAX_NG = 16
_B = 16384
_NC = 2
_NS = 16
_NW = _NC * _NS
_BPW = _B // _NW
_CHUNK = 32
_NCHUNKS = _BPW // _CHUNK
_NLANE = 16
_DV = _D // _NLANE


def _sc_body(word_ids_hbm, w_in_hbm, w_ng_hbm, ng_matrix_hbm, ng_len_hbm,
             out_hbm, idx_v, lens_v, ngids_v, ngflat_v, acc_v, ng_rows_v,
             sem_win, sem_ng, sem_len, sem_rows):
  wid = lax.axis_index("s") * _NC + lax.axis_index("c")
  base = wid * _BPW

  pltpu.sync_copy(word_ids_hbm.at[pl.ds(base, _BPW)], idx_v)
  cp_win = pltpu.async_copy(w_in_hbm.at[idx_v], acc_v, sem_win)
  cp_ng = pltpu.async_copy(ng_matrix_hbm.at[idx_v], ngids_v, sem_ng)
  cp_len = pltpu.async_copy(ng_len_hbm.at[idx_v], lens_v, sem_len)
  cp_win.wait()
  cp_ng.wait()
  cp_len.wait()

  def flat_body(w, carry):
    ngflat_v[pl.ds(w * _MAX_NG, _MAX_NG)] = ngids_v[w, pl.ds(0, _MAX_NG)]
    return carry

  lax.fori_loop(0, _BPW, flat_body, 0)

  lane = lax.iota(jnp.int32, _NLANE)
  rtab = jnp.full((_NLANE,), 1.0 / (1.0 + _MAX_NG), dtype=jnp.float32)
  for k in range(_MAX_NG - 1):
    rtab = jnp.where(lane == k, jnp.float32(1.0 / (2.0 + k)), rtab)

  def chunk_body(c, carry):
    cp_rows = pltpu.async_copy(
        w_ng_hbm.at[ngflat_v.at[pl.ds(c * _CHUNK * _MAX_NG,
                                      _CHUNK * _MAX_NG)]],
        ng_rows_v, sem_rows)
    cp_rows.wait()
    for g in range(_CHUNK // _NLANE):
      lv = lens_v[pl.ds(c * _CHUNK + g * _NLANE, _NLANE)]
      invs = jnp.take(rtab, jnp.clip(lv - 1, 0, _MAX_NG - 1), mode="fill")
      for wi in range(_NLANE):
        ws = g * _NLANE + wi
        w = c * _CHUNK + ws
        lnc = jnp.minimum(lv[wi], _MAX_NG)
        accs = tuple(acc_v[w, pl.ds(d * _NLANE, _NLANE)] for d in range(_DV))

        def j_body(j, accs, ws=ws):
          return tuple(
              accs[d] + ng_rows_v[ws * _MAX_NG + j, pl.ds(d * _NLANE, _NLANE)]
              for d in range(_DV))

        accs = lax.fori_loop(0, lnc, j_body, accs)
        inv = jnp.take(invs, jnp.full((_NLANE,), wi, dtype=jnp.int32),
                       mode="fill")
        for d in range(_DV):
          acc_v[w, pl.ds(d * _NLANE, _NLANE)] = accs[d] * inv
    return carry

  lax.fori_loop(0, _NCHUNKS, chunk_body, 0)
  pltpu.sync_copy(acc_v, out_hbm.at[pl.ds(base, _BPW)])


@jax.jit
def kernel(word_ids, W_in, W_ng, ng_matrix, ng_lengths):
  mesh = plsc.VectorSubcoreMesh(core_axis_name="c", subcore_axis_name="s")
  run = functools.partial(
      pl.kernel,
      out_type=jax.ShapeDtypeStruct((_B, _D), jnp.float32),
      mesh=mesh,
      compiler_params=pltpu.CompilerParams(use_tc_tiling_on_sc=False),
      scratch_types=[
          pltpu.VMEM((_BPW,), jnp.int32),
          pltpu.VMEM((_BPW,), jnp.int32),
          pltpu.VMEM((_BPW, _MAX_NG), jnp.int32),
          pltpu.VMEM((_BPW * _MAX_NG,), jnp.int32),
          pltpu.VMEM((_BPW, _D), jnp.float32),
          pltpu.VMEM((_CHUNK * _MAX_NG, _D), jnp.float32),
          pltpu.SemaphoreType.DMA,
          pltpu.SemaphoreType.DMA,
          pltpu.SemaphoreType.DMA,
          pltpu.SemaphoreType.DMA,
      ],
  )(_sc_body)
  return run(word_ids, W_in, W_ng, ng_matrix, ng_lengths)
